# baseline (speedup 1.0000x reference)
import functools

import jax
import jax.numpy as jnp
from jax import lax
from jax.experimental import pallas as pl
from jax.experimental.pallas import tpu as pltpu
from jax.experimental.pallas import tpu_sc as plsc

B, L, V, D = 4096, 200, 1000000, 64
DP = 128
N = B * L
NC, NS = 2, 16
NW = NC * NS
PER_W = N // NW
CHUNK = 128
NGROUPS = PER_W // CHUNK
NRING = 4
NQUADS = NGROUPS // NRING
IDX_ROWS_PER_W = PER_W // CHUNK


def _transpose_group(rows_v, tbuf):
    iota = lax.iota(jnp.int32, 16)

    @plsc.parallel_loop(0, 16, unroll=2)
    def body(s):
        perm = lax.bitwise_and(iota + s, 15)
        for jb in range(D // 16):
            colv = perm + jb * 16
            for rb in range(CHUNK // 16):
                rowv = iota + rb * 16
                vals = plsc.load_gather(rows_v, [rowv, colv])
                plsc.store_scatter(tbuf, [colv, rowv], vals)


def _make_gather():
    mesh = plsc.VectorSubcoreMesh(core_axis_name="c", subcore_axis_name="s")

    @functools.partial(
        pl.kernel,
        mesh=mesh,
        compiler_params=pltpu.CompilerParams(
            use_tc_tiling_on_sc=True, needs_layout_passes=False
        ),
        out_type=jax.ShapeDtypeStruct((D, N), jnp.float32),
        scratch_types=[
            pltpu.VMEM((IDX_ROWS_PER_W, CHUNK), jnp.int32),
            pltpu.VMEM((NRING, CHUNK, DP), jnp.float32),
            pltpu.VMEM((2, D, CHUNK), jnp.float32),
            pltpu.SemaphoreType.DMA,
            pltpu.SemaphoreType.DMA,
            pltpu.SemaphoreType.DMA,
            pltpu.SemaphoreType.DMA,
            pltpu.SemaphoreType.DMA,
            pltpu.SemaphoreType.DMA,
        ],
    )
    def gather_kernel(table_hbm, idx_hbm, out_hbm,
                      idx_all, rows_v, tbuf_v,
                      gsem0, gsem1, gsem2, gsem3, wsem0, wsem1):
        wid = lax.axis_index("s") * NC + lax.axis_index("c")
        col_base = wid * PER_W
        idx_base = wid * IDX_ROWS_PER_W
        pltpu.sync_copy(idx_hbm.at[pl.ds(idx_base, IDX_ROWS_PER_W)], idx_all)

        gsems = (gsem0, gsem1, gsem2, gsem3)
        wsems = (wsem0, wsem1)

        for q in range(NRING):
            pltpu.async_copy(
                table_hbm.at[idx_all.at[q]], rows_v.at[q], gsems[q]
            )

        def quad_body(i, carry):
            for q in range(NRING):
                g = i * NRING + q
                pltpu.make_async_copy(
                    table_hbm.at[idx_all.at[q]], rows_v.at[q], gsems[q]
                ).wait()
                @pl.when(jnp.logical_or(i > 0, q >= 2))
                def _():
                    pltpu.make_async_copy(
                        tbuf_v.at[q % 2],
                        out_hbm.at[:, pl.ds(col_base, CHUNK)],
                        wsems[q % 2],
                    ).wait()
                _transpose_group(rows_v.at[q], tbuf_v.at[q % 2])
                pltpu.async_copy(
                    tbuf_v.at[q % 2],
                    out_hbm.at[:, pl.ds(col_base + g * CHUNK, CHUNK)],
                    wsems[q % 2],
                )
                @pl.when(i < NQUADS - 1)
                def _():
                    pltpu.async_copy(
                        table_hbm.at[idx_all.at[g + NRING]],
                        rows_v.at[q],
                        gsems[q],
                    )
            return carry

        lax.fori_loop(0, NQUADS, quad_body, 0)
        for s in range(2):
            pltpu.make_async_copy(
                tbuf_v.at[s], out_hbm.at[:, pl.ds(col_base, CHUNK)], wsems[s]
            ).wait()

    return gather_kernel


_gather = _make_gather()


def kernel(nodes, nodes_mask, storage_table):
    table_pad = jnp.pad(storage_table, ((0, 0), (0, DP - D)))
    idx2d = nodes.reshape(N // CHUNK, CHUNK)
    out_t = _gather(table_pad, idx2d)
    return (out_t.T, nodes_mask)


# speedup vs baseline: 1.4288x; 1.4288x over previous
import functools

import jax
import jax.numpy as jnp
from jax import lax
from jax.experimental import pallas as pl
from jax.experimental.pallas import tpu as pltpu
from jax.experimental.pallas import tpu_sc as plsc

B, L, V, D = 4096, 200, 1000000, 64
DP = 128
N = B * L
NC, NS = 2, 16
NW = NC * NS
PER_W = N // NW
CHUNK = 128
NGROUPS = PER_W // CHUNK
NRING = 4
NQUADS = NGROUPS // NRING
IDX_ROWS_PER_W = PER_W // CHUNK

NWIN = V // CHUNK
WIN_BASE = NWIN // NW
WIN_EXTRA = NWIN % NW
TAIL = 128


def _diag_transpose(src, dst, rows, cols):
    iota = lax.iota(jnp.int32, 16)

    @plsc.parallel_loop(0, 16, unroll=2)
    def body(s):
        perm = lax.bitwise_and(iota + s, 15)
        for cb in range(cols // 16):
            colv = perm + cb * 16
            for rb in range(rows // 16):
                rowv = iota + rb * 16
                vals = plsc.load_gather(src, [rowv, colv])
                plsc.store_scatter(dst, [colv, rowv], vals)


def _make_retile():
    mesh = plsc.VectorSubcoreMesh(core_axis_name="c", subcore_axis_name="s")

    @functools.partial(
        pl.kernel,
        mesh=mesh,
        compiler_params=pltpu.CompilerParams(
            use_tc_tiling_on_sc=True, needs_layout_passes=False
        ),
        out_type=jax.ShapeDtypeStruct((V, DP), jnp.float32),
        scratch_types=[
            pltpu.VMEM((2, D, CHUNK), jnp.float32),
            pltpu.VMEM((2, CHUNK, DP), jnp.float32),
            pltpu.VMEM((TAIL, D), jnp.float32),
            pltpu.SemaphoreType.DMA,
            pltpu.SemaphoreType.DMA,
            pltpu.SemaphoreType.DMA,
            pltpu.SemaphoreType.DMA,
        ],
    )
    def retile_kernel(tt_hbm, tail_hbm, out_hbm, in_v, tbuf_v, tail_v,
                      isem0, isem1, wsem0, wsem1):
        wid = lax.axis_index("s") * NC + lax.axis_index("c")
        nwin = WIN_BASE + jnp.where(wid < WIN_EXTRA, 1, 0)
        first = wid * WIN_BASE + jnp.minimum(wid, WIN_EXTRA)

        isems = (isem0, isem1)
        wsems = (wsem0, wsem1)

        def col0_of(t):
            return t * CHUNK

        pltpu.async_copy(
            tt_hbm.at[:, pl.ds(col0_of(first), CHUNK)], in_v.at[0], isems[0]
        )
        pltpu.async_copy(
            tt_hbm.at[:, pl.ds(col0_of(first + 1), CHUNK)], in_v.at[1], isems[1]
        )

        def body(i, carry):
            s = lax.rem(i, 2)
            for sq in range(2):

                @pl.when(s == sq)
                def _():
                    col0 = col0_of(first + i)
                    pltpu.make_async_copy(
                        tt_hbm.at[:, pl.ds(col0, CHUNK)], in_v.at[sq], isems[sq]
                    ).wait()

                    @pl.when(i >= 2)
                    def _():
                        pltpu.make_async_copy(
                            tbuf_v.at[sq],
                            out_hbm.at[pl.ds(0, CHUNK)],
                            wsems[sq],
                        ).wait()

                    _diag_transpose(in_v.at[sq], tbuf_v.at[sq], D, CHUNK)
                    pltpu.async_copy(
                        tbuf_v.at[sq], out_hbm.at[pl.ds(col0, CHUNK)], wsems[sq]
                    )

                    @pl.when(i + 2 < nwin)
                    def _():
                        pltpu.async_copy(
                            tt_hbm.at[:, pl.ds(col0_of(first + i + 2), CHUNK)],
                            in_v.at[sq],
                            isems[sq],
                        )

            return carry

        lax.fori_loop(0, nwin, body, 0)
        for sq in range(2):
            pltpu.make_async_copy(
                tbuf_v.at[sq], out_hbm.at[pl.ds(0, CHUNK)], wsems[sq]
            ).wait()

        @pl.when(wid == NW - 1)
        def _():
            pltpu.sync_copy(tail_hbm, tail_v)

            @plsc.parallel_loop(0, TAIL, unroll=4)
            def _copy(r):
                for cb in range(D // 16):
                    tbuf_v[0, r, pl.ds(cb * 16, 16)] = tail_v[r, pl.ds(cb * 16, 16)]

            pltpu.sync_copy(tbuf_v.at[0], out_hbm.at[pl.ds(V - TAIL, TAIL)])

    return retile_kernel


def _make_gather():
    mesh = plsc.VectorSubcoreMesh(core_axis_name="c", subcore_axis_name="s")

    @functools.partial(
        pl.kernel,
        mesh=mesh,
        compiler_params=pltpu.CompilerParams(
            use_tc_tiling_on_sc=True, needs_layout_passes=False
        ),
        out_type=jax.ShapeDtypeStruct((D, N), jnp.float32),
        scratch_types=[
            pltpu.VMEM((IDX_ROWS_PER_W, CHUNK), jnp.int32),
            pltpu.VMEM((NRING, CHUNK, DP), jnp.float32),
            pltpu.VMEM((2, D, CHUNK), jnp.float32),
            pltpu.SemaphoreType.DMA,
            pltpu.SemaphoreType.DMA,
            pltpu.SemaphoreType.DMA,
            pltpu.SemaphoreType.DMA,
            pltpu.SemaphoreType.DMA,
            pltpu.SemaphoreType.DMA,
        ],
    )
    def gather_kernel(table_hbm, idx_hbm, out_hbm,
                      idx_all, rows_v, tbuf_v,
                      gsem0, gsem1, gsem2, gsem3, wsem0, wsem1):
        wid = lax.axis_index("s") * NC + lax.axis_index("c")
        col_base = wid * PER_W
        idx_base = wid * IDX_ROWS_PER_W
        pltpu.sync_copy(idx_hbm.at[pl.ds(idx_base, IDX_ROWS_PER_W)], idx_all)

        gsems = (gsem0, gsem1, gsem2, gsem3)
        wsems = (wsem0, wsem1)

        for q in range(NRING):
            pltpu.async_copy(
                table_hbm.at[idx_all.at[q]], rows_v.at[q], gsems[q]
            )

        def quad_body(i, carry):
            for q in range(NRING):
                g = i * NRING + q
                pltpu.make_async_copy(
                    table_hbm.at[idx_all.at[q]], rows_v.at[q], gsems[q]
                ).wait()
                @pl.when(jnp.logical_or(i > 0, q >= 2))
                def _():
                    pltpu.make_async_copy(
                        tbuf_v.at[q % 2],
                        out_hbm.at[:, pl.ds(col_base, CHUNK)],
                        wsems[q % 2],
                    ).wait()
                _diag_transpose(rows_v.at[q], tbuf_v.at[q % 2], CHUNK, D)
                pltpu.async_copy(
                    tbuf_v.at[q % 2],
                    out_hbm.at[:, pl.ds(col_base + g * CHUNK, CHUNK)],
                    wsems[q % 2],
                )
                @pl.when(i < NQUADS - 1)
                def _():
                    pltpu.async_copy(
                        table_hbm.at[idx_all.at[g + NRING]],
                        rows_v.at[q],
                        gsems[q],
                    )
            return carry

        lax.fori_loop(0, NQUADS, quad_body, 0)
        for s in range(2):
            pltpu.make_async_copy(
                tbuf_v.at[s], out_hbm.at[:, pl.ds(col_base, CHUNK)], wsems[s]
            ).wait()

    return gather_kernel


_retile = _make_retile()
_gather = _make_gather()


def kernel(nodes, nodes_mask, storage_table):
    table_rows = _retile(storage_table.T, storage_table[V - TAIL:])
    idx2d = nodes.reshape(N // CHUNK, CHUNK)
    out_t = _gather(table_rows, idx2d)
    return (out_t.T, nodes_mask)
